# baseline (speedup 1.0000x reference)
import functools

import jax
import jax.numpy as jnp
from jax import lax
from jax.experimental import pallas as pl
from jax.experimental.pallas import tpu as pltpu
from jax.experimental.pallas import tpu_sc as plsc

_CHUNK = 128


@functools.lru_cache(maxsize=None)
def _build(B, V, D):
    info = plsc.get_sparse_core_info()
    NC, NS = info.num_cores, info.num_subcores
    NW = NC * NS
    b_per_w = B // NW
    n_chunks = b_per_w // _CHUNK
    mesh = plsc.VectorSubcoreMesh(core_axis_name="c", subcore_axis_name="s")

    @functools.partial(
        pl.kernel,
        mesh=mesh,
        out_type=jax.ShapeDtypeStruct((B, D), jnp.float32),
        scratch_types=[
            pltpu.VMEM((n_chunks, _CHUNK), jnp.int32),
            pltpu.VMEM((n_chunks, _CHUNK, D), jnp.float32),
            pltpu.SemaphoreType.DMA,
        ],
        compiler_params=pltpu.CompilerParams(use_tc_tiling_on_sc=False),
    )
    def emb(idx_hbm, table_hbm, out_hbm, idx_v, rows_v, sem):
        wid = lax.axis_index("s") * NC + lax.axis_index("c")
        base = wid * b_per_w
        pltpu.sync_copy(idx_hbm.at[pl.ds(wid * n_chunks, n_chunks)], idx_v)
        descs = [
            pltpu.async_copy(table_hbm.at[idx_v.at[j]], rows_v.at[j], sem)
            for j in range(n_chunks)
        ]
        for j in range(n_chunks):
            descs[j].wait()
            pltpu.sync_copy(rows_v.at[j],
                            out_hbm.at[pl.ds(base + j * _CHUNK, _CHUNK)])

    return emb


def kernel(token_id, embedding_table):
    B = token_id.shape[0]
    V, D = embedding_table.shape
    idx2d = token_id.reshape(B // _CHUNK, _CHUNK)
    return _build(B, V, D)(idx2d, embedding_table)


# speedup vs baseline: 3.0045x; 3.0045x over previous
import functools

import jax
import jax.numpy as jnp
from jax import lax
from jax.experimental import pallas as pl
from jax.experimental.pallas import tpu as pltpu
from jax.experimental.pallas import tpu_sc as plsc

_WAVE = 16
_LANE = 16


@functools.lru_cache(maxsize=None)
def _build(B, V, D):
    info = plsc.get_sparse_core_info()
    NC, NS = info.num_cores, info.num_subcores
    NW = NC * NS
    b_per_w = B // NW
    n_waves = b_per_w // _WAVE
    mesh = plsc.VectorSubcoreMesh(core_axis_name="c", subcore_axis_name="s")

    @functools.partial(
        pl.kernel,
        mesh=mesh,
        out_type=jax.ShapeDtypeStruct((B, D), jnp.float32),
        scratch_types=[
            pltpu.VMEM((b_per_w,), jnp.int32),
            pltpu.VMEM((_WAVE // 2, D, 128), jnp.float32),
            pltpu.VMEM((b_per_w, D), jnp.float32),
            pltpu.SemaphoreType.DMA,
        ],
        compiler_params=pltpu.CompilerParams(needs_layout_passes=False),
    )
    def emb(idx_hbm, table_t_hbm, out_hbm, idx_v, blk_v, rows_v, sem):
        wid = lax.axis_index("s") * NC + lax.axis_index("c")
        base = wid * b_per_w
        pltpu.sync_copy(idx_hbm.at[pl.ds(base, b_per_w)], idx_v)
        c16 = lax.iota(jnp.int32, _LANE)

        half = _WAVE // 2

        def wave(k, _):
            ids = idx_v[pl.ds(k * _WAVE, _WAVE)]
            for hb in range(2):
                for j in range(half):
                    r = ids[hb * half + j]
                    qb = pl.multiple_of(r - (r & 127), 128)
                    pltpu.async_copy(
                        table_t_hbm.at[:, pl.ds(qb, 128)], blk_v.at[j], sem
                    )
                for j in range(half):
                    pltpu.make_async_copy(
                        table_t_hbm.at[:, pl.ds(0, 128)], blk_v.at[j], sem
                    ).wait()
                for j in range(half):
                    r = ids[hb * half + j]
                    col = jnp.full((_LANE,), r & 127, jnp.int32)
                    t = k * _WAVE + hb * half + j
                    for h in range(D // _LANE):
                        v = plsc.load_gather(
                            blk_v.at[j], [c16 + h * _LANE, col]
                        )
                        rows_v[t, pl.ds(h * _LANE, _LANE)] = v
            return ()

        lax.fori_loop(0, n_waves, wave, (), unroll=False)
        pltpu.sync_copy(rows_v, out_hbm.at[pl.ds(base, b_per_w)])

    return emb


def kernel(token_id, embedding_table):
    B = token_id.shape[0]
    V, D = embedding_table.shape
    return _build(B, V, D)(token_id, embedding_table.T)


# speedup vs baseline: 3.0664x; 1.0206x over previous
import functools

import jax
import jax.numpy as jnp
from jax import lax
from jax.experimental import pallas as pl
from jax.experimental.pallas import tpu as pltpu
from jax.experimental.pallas import tpu_sc as plsc

_WAVE = 16
_LANE = 16


@functools.lru_cache(maxsize=None)
def _build(B, V, D):
    info = plsc.get_sparse_core_info()
    NC, NS = info.num_cores, info.num_subcores
    NW = NC * NS
    b_per_w = B // NW
    n_waves = b_per_w // _WAVE
    mesh = plsc.VectorSubcoreMesh(core_axis_name="c", subcore_axis_name="s")

    @functools.partial(
        pl.kernel,
        mesh=mesh,
        out_type=jax.ShapeDtypeStruct((D, B), jnp.float32),
        scratch_types=[
            pltpu.VMEM((b_per_w,), jnp.int32),
            pltpu.VMEM((_WAVE // 2, D, 128), jnp.float32),
            pltpu.VMEM((D, b_per_w), jnp.float32),
            pltpu.SemaphoreType.DMA,
        ],
        compiler_params=pltpu.CompilerParams(needs_layout_passes=False),
    )
    def emb(idx_hbm, table_t_hbm, out_hbm, idx_v, blk_v, pan_v, sem):
        wid = lax.axis_index("s") * NC + lax.axis_index("c")
        base = wid * b_per_w
        pltpu.sync_copy(idx_hbm.at[pl.ds(base, b_per_w)], idx_v)
        c16 = lax.iota(jnp.int32, _LANE)

        half = _WAVE // 2

        def wave(k, _):
            ids = idx_v[pl.ds(k * _WAVE, _WAVE)]
            for hb in range(2):
                for j in range(half):
                    r = ids[hb * half + j]
                    qb = pl.multiple_of(r - (r & 127), 128)
                    pltpu.async_copy(
                        table_t_hbm.at[:, pl.ds(qb, 128)], blk_v.at[j], sem
                    )
                for j in range(half):
                    pltpu.make_async_copy(
                        table_t_hbm.at[:, pl.ds(0, 128)], blk_v.at[j], sem
                    ).wait()
                for j in range(half):
                    r = ids[hb * half + j]
                    col = jnp.full((_LANE,), r & 127, jnp.int32)
                    t = k * _WAVE + hb * half + j
                    tcol = jnp.full((_LANE,), t, jnp.int32)
                    for h in range(D // _LANE):
                        v = plsc.load_gather(
                            blk_v.at[j], [c16 + h * _LANE, col]
                        )
                        plsc.store_scatter(
                            pan_v, [c16 + h * _LANE, tcol], v
                        )
            return ()

        lax.fori_loop(0, n_waves, wave, (), unroll=False)
        pltpu.sync_copy(pan_v, out_hbm.at[:, pl.ds(base, b_per_w)])

    return emb


def kernel(token_id, embedding_table):
    B = token_id.shape[0]
    V, D = embedding_table.shape
    out_t = _build(B, V, D)(token_id, embedding_table.T)
    return out_t.T


# speedup vs baseline: 4.2471x; 1.3850x over previous
import functools

import jax
import jax.numpy as jnp
from jax import lax
from jax.experimental import pallas as pl
from jax.experimental.pallas import tpu as pltpu
from jax.experimental.pallas import tpu_sc as plsc

_RING = 8
_LANE = 16


@functools.lru_cache(maxsize=None)
def _build(B, V, D):
    info = plsc.get_sparse_core_info()
    NC, NS = info.num_cores, info.num_subcores
    NW = NC * NS
    b_per_w = B // NW
    n_waves = b_per_w // _RING
    mesh = plsc.VectorSubcoreMesh(core_axis_name="c", subcore_axis_name="s")

    @functools.partial(
        pl.kernel,
        mesh=mesh,
        out_type=jax.ShapeDtypeStruct((D, B), jnp.float32),
        scratch_types=[
            pltpu.VMEM((b_per_w + _LANE,), jnp.int32),
            pltpu.VMEM((_RING, D, 128), jnp.float32),
            pltpu.VMEM((D, b_per_w), jnp.float32),
        ]
        + [pltpu.SemaphoreType.DMA] * _RING,
        compiler_params=pltpu.CompilerParams(needs_layout_passes=False),
    )
    def emb(idx_hbm, table_t_hbm, out_hbm, idx_v, blk_v, pan_v, *sems):
        wid = lax.axis_index("s") * NC + lax.axis_index("c")
        base = wid * b_per_w
        pltpu.sync_copy(idx_hbm.at[pl.ds(base, b_per_w)],
                        idx_v.at[pl.ds(0, b_per_w)])
        c16 = lax.iota(jnp.int32, _LANE)

        def fire(r, j):
            qb = pl.multiple_of(r - (r & 127), 128)
            pltpu.async_copy(
                table_t_hbm.at[:, pl.ds(qb, 128)], blk_v.at[j], sems[j]
            )

        def drain(j):
            pltpu.make_async_copy(
                table_t_hbm.at[:, pl.ds(0, 128)], blk_v.at[j], sems[j]
            ).wait()

        def extract(r, t, j):
            col = jnp.full((_LANE,), r & 127, jnp.int32)
            tcol = jnp.full((_LANE,), t, jnp.int32)
            for h in range(D // _LANE):
                v = plsc.load_gather(blk_v.at[j], [c16 + h * _LANE, col])
                plsc.store_scatter(pan_v, [c16 + h * _LANE, tcol], v)

        ids0 = idx_v[pl.ds(0, _LANE)]
        for j in range(_RING):
            fire(ids0[j], j)

        def wave(k, _):
            ids = idx_v[pl.ds((k - 1) * _RING, _LANE)]
            for j in range(_RING):
                drain(j)
                extract(ids[j], (k - 1) * _RING + j, j)

                @pl.when(k < n_waves)
                def _():
                    fire(ids[_RING + j], j)

            return ()

        lax.fori_loop(1, n_waves + 1, wave, (), unroll=False)
        pltpu.sync_copy(pan_v, out_hbm.at[:, pl.ds(base, b_per_w)])

    return emb


def kernel(token_id, embedding_table):
    B = token_id.shape[0]
    V, D = embedding_table.shape
    out_t = _build(B, V, D)(token_id, embedding_table.T)
    return out_t.T
